# baseline (speedup 1.0000x reference)
import jax
import jax.numpy as jnp
from jax import lax
from jax.experimental import pallas as pl
from jax.experimental.pallas import tpu as pltpu, tpu_sc as plsc

_N = 100000
_E = 1600000
_G = 256
_LEAK = 0.1

_NP = 100352
_NB = 98
_BK = 1024
_NW = 32
_ES = _E // _NW
_RS = 50688
_PACKED = _NW * _RS + 512
_HB = 112
_MROWS = _E + 64
_ACCR = 1152
_ZR = _ACCR // 16
_GW = 2000
_CW = 2000
_VW = 1024
_EB = 8000
_NBK = 1024

_BN_S = (1.0 + 1e-5) ** -0.5


def _leaky(v):
    return jnp.where(v > 0, v, _LEAK * v)


def _wid():
    return lax.axis_index("s") * 2 + lax.axis_index("c")


def _k1_body(x16, row, col, gsrc, gdst, idx_v, rows_v, sem):
    w = _wid()
    base0 = w * _ES

    def win(i, carry):
        base = pl.multiple_of(base0 + i * _GW, 8)
        pltpu.sync_copy(row.at[pl.ds(base, _GW)], idx_v)
        pltpu.async_copy(x16.at[idx_v], rows_v, sem).wait()
        pltpu.sync_copy(rows_v, gsrc.at[pl.ds(base, _GW)])
        pltpu.sync_copy(col.at[pl.ds(base, _GW)], idx_v)
        pltpu.async_copy(x16.at[idx_v], rows_v, sem).wait()
        pltpu.sync_copy(rows_v, gdst.at[pl.ds(base, _GW)])
        return carry

    lax.fori_loop(0, _ES // _GW, win, 0)


def _st1(ref, o, val):
    idx = jnp.broadcast_to(o, (16,))
    v = jnp.broadcast_to(val, (16,))
    plsc.store_scatter(ref, [idx], v, mask=lax.iota(jnp.int32, 16) == 0)


def _k3a_body(col, packed, hist, colv, hstage, outw, histv, curv, basev):
    w = _wid()
    base0 = w * _ES

    def zero(i, carry):
        histv[i] = 0
        return carry

    lax.fori_loop(0, _HB, zero, 0)

    def hwin(i, carry):
        pltpu.sync_copy(col.at[pl.ds(pl.multiple_of(base0 + i * _CW, 8), _CW)], colv)

        def inner(v, c2):
            bv = colv[pl.ds(v * 16, 16)] >> 10
            for l in range(16):
                b = bv[l]
                histv[b] = histv[b] + 1
            return c2

        lax.fori_loop(0, _CW // 16, inner, 0)
        return carry

    lax.fori_loop(0, _ES // _CW, hwin, 0)

    def pfx(b, run):
        basev[b] = run
        curv[b] = run
        _st1(hstage, b, histv[b])
        return run + ((histv[b] + 7) & (-8))

    lax.fori_loop(0, _NB, pfx, 0)

    def pwin(i, carry):
        wbase = base0 + i * _CW
        pltpu.sync_copy(col.at[pl.ds(pl.multiple_of(wbase, 8), _CW)], colv)

        def inner(v, c2):
            cv = colv[pl.ds(v * 16, 16)]
            wordv = (((wbase + v * 16 + lax.iota(jnp.int32, 16)) << 11)
                     | (cv & 1023))
            bv = cv >> 10
            for l in range(16):
                b = bv[l]
                o = curv[b]
                _st1(outw, o, wordv[l])
                curv[b] = o + 1
            return c2

        lax.fori_loop(0, _CW // 16, inner, 0)
        return carry

    lax.fori_loop(0, _ES // _CW, pwin, 0)

    dump = ((_E + w) << 11) | 1024

    def pad(b, carry):
        end = basev[b] + ((histv[b] + 7) & (-8))

        def fill(k2, c2):
            _st1(outw, k2, dump)
            return c2

        lax.fori_loop(curv[b], end, fill, 0)
        return carry

    lax.fori_loop(0, _NB, pad, 0)

    pltpu.sync_copy(outw, packed.at[pl.ds(w * _RS, _RS)])
    pltpu.sync_copy(hstage, hist.at[pl.ds(w * _HB, _HB)])


def _k3b_body(packed, hist, m, msum, mcnt,
              chunkv, idxv, offv, histv, mrows, zrow, cntp, idv,
              acc_sh, cnt_sh, sem):
    c = lax.axis_index("c")
    sid = lax.axis_index("s")
    pltpu.sync_copy(hist, histv)
    dump_eid = _E + _wid()
    dump_row = _BK + sid
    zero16 = jnp.zeros((16,), jnp.float32)
    iot16 = lax.iota(jnp.int32, 16)

    def initbufs(r, carry):
        zrow[r, pl.ds(0, 16)] = zero16
        zrow[r, pl.ds(16, 16)] = zero16
        zrow[r, pl.ds(32, 16)] = zero16
        zrow[r, pl.ds(48, 16)] = zero16
        return carry

    lax.fori_loop(0, _ZR, initbufs, 0)

    def initid(r, carry):
        idv[pl.ds(r * 16, 16)] = iot16 + r * 16
        return carry

    lax.fori_loop(0, _ACCR // 16, initid, 0)

    def bucket(k, carry):
        b = c + 2 * k
        zb = pl.multiple_of(sid * (_ACCR // 16), 8)
        pltpu.sync_copy(zrow.at[pl.ds(0, _ACCR // 16)],
                        acc_sh.at[pl.ds(zb, _ACCR // 16)])
        def zcnt(r, c2):
            cntp[pl.ds(r * 16, 16)] = zero16
            return c2

        lax.fori_loop(0, _ACCR // 16, zcnt, 0)
        pltpu.sync_copy(cntp.at[pl.ds(0, _ACCR // 16)],
                        cnt_sh.at[pl.ds(zb, _ACCR // 16)])
        plsc.subcore_barrier()

        for j in range(2):
            t = sid + 16 * j
            iot = lax.iota(jnp.int32, 16)
            segoff = 0
            seglen = 0
            for vi in range(_HB // 16):
                hv = histv[pl.ds(t * _HB + vi * 16, 16)]
                idx = iot + vi * 16
                r8 = (hv + 7) & (-8)
                segoff = segoff + jnp.sum(jnp.where(idx < b, r8, 0))
                seglen = seglen + jnp.sum(jnp.where(idx == b, r8, 0))
            segstart = pl.multiple_of(t * _RS + segoff, 8)
            nwin = (seglen + _VW - 1) // _VW

            def win(kk, c4):
                s0 = pl.multiple_of(segstart + kk * _VW, 8)
                pltpu.sync_copy(packed.at[pl.ds(s0, _VW)], chunkv)
                limit = seglen - kk * _VW
                for i in range(_VW // 16):
                    wv = plsc.bitcast(chunkv[pl.ds(i * 16, 16)], jnp.uint32)
                    pos = lax.iota(jnp.int32, 16) + (i * 16)
                    realm = pos < limit
                    eidv = jnp.where(
                        realm,
                        plsc.bitcast(wv >> jnp.uint32(11), jnp.int32),
                        dump_eid)
                    ofv = jnp.where(
                        realm,
                        plsc.bitcast(wv & jnp.uint32(2047), jnp.int32),
                        dump_row)
                    plsc.addupdate_scatter(cntp, [ofv],
                                           jnp.ones((16,), jnp.float32))
                    idxv[pl.ds(i * 16, 16)] = eidv
                    offv[pl.ds(i * 16, 16)] = ofv
                pltpu.async_copy(m.at[idxv], mrows, sem).wait()
                pltpu.sync_copy(mrows, acc_sh.at[offv], add=True)
                return c4

            lax.fori_loop(0, nwin, win, 0)

        pltpu.sync_copy(cntp, cnt_sh.at[idv], add=True)
        plsc.subcore_barrier()
        wb = pl.multiple_of(sid * (_BK // 16), 8)
        pltpu.sync_copy(acc_sh.at[pl.ds(wb, _BK // 16)],
                        msum.at[pl.ds(b * _BK + wb, _BK // 16)])
        pltpu.sync_copy(cnt_sh.at[pl.ds(wb, _BK // 16)],
                        mcnt.at[pl.ds(b * _BK + wb, _BK // 16)])
        plsc.subcore_barrier()
        return carry

    lax.fori_loop(0, _NB // 2, bucket, 0)


def _k2_body(gs, gd, ea, wc, mo):
    w = wc[...]
    a = jnp.dot(gs[...], w[0:16], preferred_element_type=jnp.float32)
    a = a + jnp.dot(gd[...], w[16:32], preferred_element_type=jnp.float32)
    a = a + jnp.dot(ea[...], w[32:44], preferred_element_type=jnp.float32)
    a = _leaky(a + w[384:385])
    a = _leaky(jnp.dot(a, w[48:112], preferred_element_type=jnp.float32)
               + w[385:386])
    e = jnp.dot(a, w[112:176], preferred_element_type=jnp.float32) + w[386:387]
    h = jnp.dot(gs[...], w[176:192], preferred_element_type=jnp.float32)
    h = h + jnp.dot(e, w[192:256], preferred_element_type=jnp.float32)
    h = _leaky(h + w[387:388])
    h = _leaky(jnp.dot(h, w[256:320], preferred_element_type=jnp.float32)
               + w[388:389])
    mo[...] = (jnp.dot(h, w[320:384], preferred_element_type=jnp.float32)
               + w[389:390])


def _k4_body(xb, ms, mc, bf, wc, out):
    i = pl.program_id(0)
    w = wc[...]
    cntc = jnp.maximum(mc[...][:, 0:1], 1.0)
    agg = ms[...] / cntc
    h = jnp.dot(xb[...], w[0:16], preferred_element_type=jnp.float32)
    h = h + jnp.dot(agg, w[16:80], preferred_element_type=jnp.float32)
    h = _leaky(h + w[208:209])
    h = _leaky(jnp.dot(h, w[80:144], preferred_element_type=jnp.float32)
               + w[209:210])
    xn = jnp.dot(h, w[144:208], preferred_element_type=jnp.float32) + w[210:211]
    oh = (bf[...] ==
          lax.broadcasted_iota(jnp.int32, (_NBK, _G), 1).astype(jnp.float32))
    oh = oh.astype(jnp.float32)
    xaug = jnp.concatenate(
        [xn, jnp.ones((_NBK, 1), jnp.float32),
         jnp.zeros((_NBK, 63), jnp.float32)], axis=1)
    contrib = lax.dot_general(oh, xaug, (((0,), (0,)), ((), ())),
                              preferred_element_type=jnp.float32)

    @pl.when(i == 0)
    def _():
        out[...] = jnp.zeros_like(out)

    out[...] = out[...] + contrib


def _k5_body(ps, gw, ht, hb, out):
    p = ps[...]
    wg = gw[...]
    u = p[:, 0:64] / jnp.maximum(p[:, 64:65], 1.0)
    u = _leaky(jnp.dot(u, wg[0:64], preferred_element_type=jnp.float32)
               + wg[192:193])
    u = _leaky(jnp.dot(u, wg[64:128], preferred_element_type=jnp.float32)
               + wg[193:194])
    u = jnp.dot(u, wg[128:192], preferred_element_type=jnp.float32) + wg[194:195]
    z = jnp.dot(u, ht[...], preferred_element_type=jnp.float32) + hb[...][0:1]
    cols = [jax.nn.sigmoid(z[:, 0:1])]
    for gidx in range(6):
        zg = z[:, 1 + 4 * gidx:5 + 4 * gidx]
        zm = jnp.max(zg, axis=1, keepdims=True)
        ez = jnp.exp(zg - zm)
        cols.append(ez / jnp.sum(ez, axis=1, keepdims=True))
    cols.append(jnp.zeros((_G, 128 - 25), jnp.float32))
    out[...] = jnp.concatenate(cols, axis=1)


def _fold(params):
    out = []
    for (g, b, wt, c) in params:
        wl = (wt * (g * _BN_S)[None, :]).T
        out.append((wl, b @ wt.T + c))
    return out


def _z16(a):
    return jnp.pad(a, ((0, 16 - a.shape[0]), (0, 0)))


def kernel(x, edge_index, edge_attr, batch, edge_mlp, node_mlp1, node_mlp2,
           global_mlp, preds):
    f32 = jnp.float32
    row = edge_index[0]
    col = edge_index[1]
    x16 = jnp.pad(x, ((0, _NP - _N), (0, 16 - x.shape[1])))
    batchf = jnp.pad(batch.astype(f32), ((0, _NP - _N),),
                     constant_values=300.0).reshape(_NP, 1)

    em = _fold(edge_mlp)
    n1 = _fold(node_mlp1)
    n2 = _fold(node_mlp2)
    gl = _fold(global_mlp)
    (w1, c1), (w2, c2), (w3, c3) = em
    (b1, d1), (b2, d2), (b3, d3) = n1
    wcat = jnp.concatenate([
        _z16(w1[0:9]), _z16(w1[9:18]), w1[18:30], jnp.zeros((4, 64), f32),
        w2, w3, _z16(b1[0:9]), b1[9:73], b2, b3,
        c1[None], c2[None], c3[None], d1[None], d2[None], d3[None],
        jnp.zeros((2, 64), f32)], axis=0)
    (a1, e1), (a2, e2), (a3, e3) = n2
    wcat2 = jnp.concatenate([
        _z16(a1[0:9]), a1[9:73], a2, a3,
        e1[None], e2[None], e3[None], jnp.zeros((5, 64), f32)], axis=0)
    (g1, f1), (g2, f2), (g3, f3) = gl
    gwcat = jnp.concatenate([
        g1, g2, g3, f1[None], f2[None], f3[None],
        jnp.zeros((5, 64), f32)], axis=0)
    wh = jnp.concatenate([p[0] for p in preds], axis=0)
    ht = jnp.pad(wh.T, ((0, 0), (0, 7)))
    hbv = jnp.pad(jnp.concatenate([p[1] for p in preds]), ((0, 7),))
    hb = jnp.pad(hbv[None], ((0, 7), (0, 0)))

    mesh = plsc.VectorSubcoreMesh(core_axis_name="c", subcore_axis_name="s")


    packed, hist = pl.kernel(
        _k3a_body,
        out_type=(jax.ShapeDtypeStruct((_PACKED,), jnp.int32),
                  jax.ShapeDtypeStruct((_NW * _HB,), jnp.int32)),
        mesh=mesh,
        scratch_types=[pltpu.VMEM((_CW,), jnp.int32),
                       pltpu.VMEM((_HB,), jnp.int32),
                       pltpu.VMEM((_RS,), jnp.int32),
                       pltpu.SMEM((_HB,), jnp.int32),
                       pltpu.SMEM((_HB,), jnp.int32),
                       pltpu.SMEM((_HB,), jnp.int32)],
        compiler_params=pltpu.CompilerParams(needs_layout_passes=False),
    )(col)

    gsrc, gdst = pl.kernel(
        _k1_body,
        out_type=(jax.ShapeDtypeStruct((_E, 16), f32),
                  jax.ShapeDtypeStruct((_E, 16), f32)),
        mesh=mesh,
        scratch_types=[pltpu.VMEM((_GW,), jnp.int32),
                       pltpu.VMEM((_GW, 16), f32),
                       pltpu.SemaphoreType.DMA],
        compiler_params=pltpu.CompilerParams(use_tc_tiling_on_sc=False),
    )(x16, row, col)

    m = pl.pallas_call(
        _k2_body,
        grid=(_E // _EB,),
        in_specs=[pl.BlockSpec((_EB, 16), lambda i: (i, 0)),
                  pl.BlockSpec((_EB, 16), lambda i: (i, 0)),
                  pl.BlockSpec((_EB, 12), lambda i: (i, 0)),
                  pl.BlockSpec((392, 64), lambda i: (0, 0))],
        out_specs=pl.BlockSpec((_EB, 64), lambda i: (i, 0)),
        out_shape=jax.ShapeDtypeStruct((_MROWS, 64), f32),
    )(gsrc, gdst, edge_attr, wcat)

    msum, mcnt = pl.kernel(
        _k3b_body,
        out_type=(jax.ShapeDtypeStruct((_NP, 64), f32),
                  jax.ShapeDtypeStruct((_NP,), f32)),
        mesh=mesh,
        scratch_types=[pltpu.VMEM((_VW,), jnp.int32),
                       pltpu.VMEM((_VW,), jnp.int32),
                       pltpu.VMEM((_VW,), jnp.int32),
                       pltpu.VMEM((_NW * _HB,), jnp.int32),
                       pltpu.VMEM((_VW, 64), f32),
                       pltpu.VMEM((_ZR, 64), f32),
                       pltpu.VMEM((_ACCR,), f32),
                       pltpu.VMEM((_ACCR,), jnp.int32),
                       pltpu.VMEM_SHARED((_ACCR, 64), f32),
                       pltpu.VMEM_SHARED((_ACCR,), f32),
                       pltpu.SemaphoreType.DMA],
        compiler_params=pltpu.CompilerParams(use_tc_tiling_on_sc=False,
                                             needs_layout_passes=False),
    )(packed, hist, m)

    psum = pl.pallas_call(
        _k4_body,
        grid=(_NP // _NBK,),
        in_specs=[pl.BlockSpec((_NBK, 16), lambda i: (i, 0)),
                  pl.BlockSpec((_NBK, 64), lambda i: (i, 0)),
                  pl.BlockSpec((_NBK, 1), lambda i: (i, 0)),
                  pl.BlockSpec((_NBK, 1), lambda i: (i, 0)),
                  pl.BlockSpec((216, 64), lambda i: (0, 0))],
        out_specs=pl.BlockSpec((_G, 128), lambda i: (0, 0)),
        out_shape=jax.ShapeDtypeStruct((_G, 128), f32),
    )(x16, msum, mcnt.reshape(_NP, 1), batchf, wcat2)

    out = pl.pallas_call(
        _k5_body,
        grid=(1,),
        in_specs=[pl.BlockSpec((_G, 128), lambda i: (0, 0)),
                  pl.BlockSpec((200, 64), lambda i: (0, 0)),
                  pl.BlockSpec((64, 32), lambda i: (0, 0)),
                  pl.BlockSpec((8, 32), lambda i: (0, 0))],
        out_specs=pl.BlockSpec((_G, 128), lambda i: (0, 0)),
        out_shape=jax.ShapeDtypeStruct((_G, 128), f32),
    )(psum, gwcat, ht, hb)

    return (out[:, 0:1], out[:, 1:5], out[:, 5:9], out[:, 9:13],
            out[:, 13:17], out[:, 17:21], out[:, 21:25])


# speedup vs baseline: 1.2148x; 1.2148x over previous
import jax
import jax.numpy as jnp
from jax import lax
from jax.experimental import pallas as pl
from jax.experimental.pallas import tpu as pltpu, tpu_sc as plsc

_N = 100000
_E = 1600000
_G = 256
_LEAK = 0.1

_NP = 100352
_NB = 98
_BK = 1024
_NW = 32
_ES = _E // _NW
_RS = 50688
_PACKED = _NW * _RS + 512
_HB = 112
_MROWS = _E + 64
_ACCR = 1152
_ZR = _ACCR // 16
_GW = 2000
_CW = 2000
_VW = 256
_EB = 8000
_NBK = 1024

_BN_S = (1.0 + 1e-5) ** -0.5


def _leaky(v):
    return jnp.where(v > 0, v, _LEAK * v)


def _wid():
    return lax.axis_index("s") * 2 + lax.axis_index("c")


def _k1_body(x16, row, col, gsrc, gdst, idx_v, rows_v, sem):
    w = _wid()
    base0 = w * _ES

    def win(i, carry):
        base = pl.multiple_of(base0 + i * _GW, 8)
        pltpu.sync_copy(row.at[pl.ds(base, _GW)], idx_v)
        pltpu.async_copy(x16.at[idx_v], rows_v, sem).wait()
        pltpu.sync_copy(rows_v, gsrc.at[pl.ds(base, _GW)])
        pltpu.sync_copy(col.at[pl.ds(base, _GW)], idx_v)
        pltpu.async_copy(x16.at[idx_v], rows_v, sem).wait()
        pltpu.sync_copy(rows_v, gdst.at[pl.ds(base, _GW)])
        return carry

    lax.fori_loop(0, _ES // _GW, win, 0)


def _st1(ref, o, val):
    idx = jnp.broadcast_to(o, (16,))
    v = jnp.broadcast_to(val, (16,))
    plsc.store_scatter(ref, [idx], v, mask=lax.iota(jnp.int32, 16) == 0)


def _k3a_body(col, packed, hist, colv, hstage, outw, histv, curv, basev):
    w = _wid()
    base0 = w * _ES

    def zero(i, carry):
        histv[i] = 0
        return carry

    lax.fori_loop(0, _HB, zero, 0)

    def hwin(i, carry):
        pltpu.sync_copy(col.at[pl.ds(pl.multiple_of(base0 + i * _CW, 8), _CW)], colv)

        def inner(v, c2):
            bv = colv[pl.ds(v * 16, 16)] >> 10
            for l in range(16):
                b = bv[l]
                histv[b] = histv[b] + 1
            return c2

        lax.fori_loop(0, _CW // 16, inner, 0)
        return carry

    lax.fori_loop(0, _ES // _CW, hwin, 0)

    def pfx(b, run):
        basev[b] = run
        curv[b] = run
        _st1(hstage, b, histv[b])
        return run + ((histv[b] + 7) & (-8))

    lax.fori_loop(0, _NB, pfx, 0)

    def pwin(i, carry):
        wbase = base0 + i * _CW
        pltpu.sync_copy(col.at[pl.ds(pl.multiple_of(wbase, 8), _CW)], colv)

        def inner(v, c2):
            cv = colv[pl.ds(v * 16, 16)]
            wordv = (((wbase + v * 16 + lax.iota(jnp.int32, 16)) << 11)
                     | (cv & 1023))
            bv = cv >> 10
            for l in range(16):
                b = bv[l]
                o = curv[b]
                _st1(outw, o, wordv[l])
                curv[b] = o + 1
            return c2

        lax.fori_loop(0, _CW // 16, inner, 0)
        return carry

    lax.fori_loop(0, _ES // _CW, pwin, 0)

    dump = ((_E + w) << 11) | 1024

    def pad(b, carry):
        end = basev[b] + ((histv[b] + 7) & (-8))

        def fill(k2, c2):
            _st1(outw, k2, dump)
            return c2

        lax.fori_loop(curv[b], end, fill, 0)
        return carry

    lax.fori_loop(0, _NB, pad, 0)

    pltpu.sync_copy(outw, packed.at[pl.ds(w * _RS, _RS)])
    pltpu.sync_copy(hstage, hist.at[pl.ds(w * _HB, _HB)])


def _k3b_body(packed, hist, m, msum, mcnt,
              chunkv, idxv, offv, histv, mrows, zrow, cntp, idv,
              acc_sh, cnt_sh, sem):
    c = lax.axis_index("c")
    sid = lax.axis_index("s")
    pltpu.sync_copy(hist, histv)
    dump_eid = _E + _wid()
    dump_row = _BK + sid
    zero16 = jnp.zeros((16,), jnp.float32)
    iot16 = lax.iota(jnp.int32, 16)

    def initbufs(r, carry):
        zrow[r, pl.ds(0, 16)] = zero16
        zrow[r, pl.ds(16, 16)] = zero16
        zrow[r, pl.ds(32, 16)] = zero16
        zrow[r, pl.ds(48, 16)] = zero16
        return carry

    lax.fori_loop(0, _ZR, initbufs, 0)

    def initid(r, carry):
        idv[pl.ds(r * 16, 16)] = iot16 + r * 16
        return carry

    lax.fori_loop(0, _ACCR // 16, initid, 0)

    def bucket(k, carry):
        b = c + 2 * k
        zb = pl.multiple_of(sid * (_ACCR // 16), 8)
        pltpu.sync_copy(zrow.at[pl.ds(0, _ACCR // 16)],
                        acc_sh.at[pl.ds(zb, _ACCR // 16)])
        def zcnt(r, c2):
            cntp[pl.ds(r * 16, 16)] = zero16
            return c2

        lax.fori_loop(0, _ACCR // 16, zcnt, 0)
        pltpu.sync_copy(cntp.at[pl.ds(0, _ACCR // 16)],
                        cnt_sh.at[pl.ds(zb, _ACCR // 16)])
        plsc.subcore_barrier()

        for j in range(2):
            t = sid + 16 * j
            iot = lax.iota(jnp.int32, 16)
            segoff = 0
            seglen = 0
            for vi in range(_HB // 16):
                hv = histv[pl.ds(t * _HB + vi * 16, 16)]
                idx = iot + vi * 16
                r8 = (hv + 7) & (-8)
                segoff = segoff + jnp.sum(jnp.where(idx < b, r8, 0))
                seglen = seglen + jnp.sum(jnp.where(idx == b, r8, 0))
            segstart = pl.multiple_of(t * _RS + segoff, 8)
            nwin = (seglen + _VW - 1) // _VW

            def win(kk, c4):
                s0 = pl.multiple_of(segstart + kk * _VW, 8)
                pltpu.sync_copy(packed.at[pl.ds(s0, _VW)], chunkv)
                limit = seglen - kk * _VW
                for i in range(_VW // 16):
                    wv = plsc.bitcast(chunkv[pl.ds(i * 16, 16)], jnp.uint32)
                    pos = lax.iota(jnp.int32, 16) + (i * 16)
                    realm = pos < limit
                    eidv = jnp.where(
                        realm,
                        plsc.bitcast(wv >> jnp.uint32(11), jnp.int32),
                        dump_eid)
                    ofv = jnp.where(
                        realm,
                        plsc.bitcast(wv & jnp.uint32(2047), jnp.int32),
                        dump_row)
                    plsc.addupdate_scatter(cntp, [ofv],
                                           jnp.ones((16,), jnp.float32))
                    idxv[pl.ds(i * 16, 16)] = eidv
                    offv[pl.ds(i * 16, 16)] = ofv
                pltpu.async_copy(m.at[idxv], mrows, sem).wait()
                pltpu.sync_copy(mrows, acc_sh.at[offv], add=True)
                return c4

            lax.fori_loop(0, nwin, win, 0)

        pltpu.sync_copy(cntp, cnt_sh.at[idv], add=True)
        plsc.subcore_barrier()
        wb = pl.multiple_of(sid * (_BK // 16), 8)
        pltpu.sync_copy(acc_sh.at[pl.ds(wb, _BK // 16)],
                        msum.at[pl.ds(b * _BK + wb, _BK // 16)])
        pltpu.sync_copy(cnt_sh.at[pl.ds(wb, _BK // 16)],
                        mcnt.at[pl.ds(b * _BK + wb, _BK // 16)])
        plsc.subcore_barrier()
        return carry

    lax.fori_loop(0, _NB // 2, bucket, 0)


def _k2_body(gs, gd, ea, wc, mo):
    w = wc[...]
    a = jnp.dot(gs[...], w[0:16], preferred_element_type=jnp.float32)
    a = a + jnp.dot(gd[...], w[16:32], preferred_element_type=jnp.float32)
    a = a + jnp.dot(ea[...], w[32:44], preferred_element_type=jnp.float32)
    a = _leaky(a + w[384:385])
    a = _leaky(jnp.dot(a, w[48:112], preferred_element_type=jnp.float32)
               + w[385:386])
    e = jnp.dot(a, w[112:176], preferred_element_type=jnp.float32) + w[386:387]
    h = jnp.dot(gs[...], w[176:192], preferred_element_type=jnp.float32)
    h = h + jnp.dot(e, w[192:256], preferred_element_type=jnp.float32)
    h = _leaky(h + w[387:388])
    h = _leaky(jnp.dot(h, w[256:320], preferred_element_type=jnp.float32)
               + w[388:389])
    mo[...] = (jnp.dot(h, w[320:384], preferred_element_type=jnp.float32)
               + w[389:390])


def _k4_body(xb, ms, mc, bf, wc, out):
    i = pl.program_id(0)
    w = wc[...]
    cntc = jnp.maximum(mc[...][:, 0:1], 1.0)
    agg = ms[...] / cntc
    h = jnp.dot(xb[...], w[0:16], preferred_element_type=jnp.float32)
    h = h + jnp.dot(agg, w[16:80], preferred_element_type=jnp.float32)
    h = _leaky(h + w[208:209])
    h = _leaky(jnp.dot(h, w[80:144], preferred_element_type=jnp.float32)
               + w[209:210])
    xn = jnp.dot(h, w[144:208], preferred_element_type=jnp.float32) + w[210:211]
    oh = (bf[...] ==
          lax.broadcasted_iota(jnp.int32, (_NBK, _G), 1).astype(jnp.float32))
    oh = oh.astype(jnp.float32)
    xaug = jnp.concatenate(
        [xn, jnp.ones((_NBK, 1), jnp.float32),
         jnp.zeros((_NBK, 63), jnp.float32)], axis=1)
    contrib = lax.dot_general(oh, xaug, (((0,), (0,)), ((), ())),
                              preferred_element_type=jnp.float32)

    @pl.when(i == 0)
    def _():
        out[...] = jnp.zeros_like(out)

    out[...] = out[...] + contrib


def _k5_body(ps, gw, ht, hb, out):
    p = ps[...]
    wg = gw[...]
    u = p[:, 0:64] / jnp.maximum(p[:, 64:65], 1.0)
    u = _leaky(jnp.dot(u, wg[0:64], preferred_element_type=jnp.float32)
               + wg[192:193])
    u = _leaky(jnp.dot(u, wg[64:128], preferred_element_type=jnp.float32)
               + wg[193:194])
    u = jnp.dot(u, wg[128:192], preferred_element_type=jnp.float32) + wg[194:195]
    z = jnp.dot(u, ht[...], preferred_element_type=jnp.float32) + hb[...][0:1]
    cols = [jax.nn.sigmoid(z[:, 0:1])]
    for gidx in range(6):
        zg = z[:, 1 + 4 * gidx:5 + 4 * gidx]
        zm = jnp.max(zg, axis=1, keepdims=True)
        ez = jnp.exp(zg - zm)
        cols.append(ez / jnp.sum(ez, axis=1, keepdims=True))
    cols.append(jnp.zeros((_G, 128 - 25), jnp.float32))
    out[...] = jnp.concatenate(cols, axis=1)


def _fold(params):
    out = []
    for (g, b, wt, c) in params:
        wl = (wt * (g * _BN_S)[None, :]).T
        out.append((wl, b @ wt.T + c))
    return out


def _z16(a):
    return jnp.pad(a, ((0, 16 - a.shape[0]), (0, 0)))


def kernel(x, edge_index, edge_attr, batch, edge_mlp, node_mlp1, node_mlp2,
           global_mlp, preds):
    f32 = jnp.float32
    row = edge_index[0]
    col = edge_index[1]
    x16 = jnp.pad(x, ((0, _NP - _N), (0, 16 - x.shape[1])))
    batchf = jnp.pad(batch.astype(f32), ((0, _NP - _N),),
                     constant_values=300.0).reshape(_NP, 1)

    em = _fold(edge_mlp)
    n1 = _fold(node_mlp1)
    n2 = _fold(node_mlp2)
    gl = _fold(global_mlp)
    (w1, c1), (w2, c2), (w3, c3) = em
    (b1, d1), (b2, d2), (b3, d3) = n1
    wcat = jnp.concatenate([
        _z16(w1[0:9]), _z16(w1[9:18]), w1[18:30], jnp.zeros((4, 64), f32),
        w2, w3, _z16(b1[0:9]), b1[9:73], b2, b3,
        c1[None], c2[None], c3[None], d1[None], d2[None], d3[None],
        jnp.zeros((2, 64), f32)], axis=0)
    (a1, e1), (a2, e2), (a3, e3) = n2
    wcat2 = jnp.concatenate([
        _z16(a1[0:9]), a1[9:73], a2, a3,
        e1[None], e2[None], e3[None], jnp.zeros((5, 64), f32)], axis=0)
    (g1, f1), (g2, f2), (g3, f3) = gl
    gwcat = jnp.concatenate([
        g1, g2, g3, f1[None], f2[None], f3[None],
        jnp.zeros((5, 64), f32)], axis=0)
    wh = jnp.concatenate([p[0] for p in preds], axis=0)
    ht = jnp.pad(wh.T, ((0, 0), (0, 7)))
    hbv = jnp.pad(jnp.concatenate([p[1] for p in preds]), ((0, 7),))
    hb = jnp.pad(hbv[None], ((0, 7), (0, 0)))

    mesh = plsc.VectorSubcoreMesh(core_axis_name="c", subcore_axis_name="s")


    packed, hist = pl.kernel(
        _k3a_body,
        out_type=(jax.ShapeDtypeStruct((_PACKED,), jnp.int32),
                  jax.ShapeDtypeStruct((_NW * _HB,), jnp.int32)),
        mesh=mesh,
        scratch_types=[pltpu.VMEM((_CW,), jnp.int32),
                       pltpu.VMEM((_HB,), jnp.int32),
                       pltpu.VMEM((_RS,), jnp.int32),
                       pltpu.SMEM((_HB,), jnp.int32),
                       pltpu.SMEM((_HB,), jnp.int32),
                       pltpu.SMEM((_HB,), jnp.int32)],
        compiler_params=pltpu.CompilerParams(needs_layout_passes=False),
    )(col)

    gsrc, gdst = pl.kernel(
        _k1_body,
        out_type=(jax.ShapeDtypeStruct((_E, 16), f32),
                  jax.ShapeDtypeStruct((_E, 16), f32)),
        mesh=mesh,
        scratch_types=[pltpu.VMEM((_GW,), jnp.int32),
                       pltpu.VMEM((_GW, 16), f32),
                       pltpu.SemaphoreType.DMA],
        compiler_params=pltpu.CompilerParams(use_tc_tiling_on_sc=False),
    )(x16, row, col)

    m = pl.pallas_call(
        _k2_body,
        grid=(_E // _EB,),
        in_specs=[pl.BlockSpec((_EB, 16), lambda i: (i, 0)),
                  pl.BlockSpec((_EB, 16), lambda i: (i, 0)),
                  pl.BlockSpec((_EB, 12), lambda i: (i, 0)),
                  pl.BlockSpec((392, 64), lambda i: (0, 0))],
        out_specs=pl.BlockSpec((_EB, 64), lambda i: (i, 0)),
        out_shape=jax.ShapeDtypeStruct((_MROWS, 64), f32),
    )(gsrc, gdst, edge_attr, wcat)

    msum, mcnt = pl.kernel(
        _k3b_body,
        out_type=(jax.ShapeDtypeStruct((_NP, 64), f32),
                  jax.ShapeDtypeStruct((_NP,), f32)),
        mesh=mesh,
        scratch_types=[pltpu.VMEM((_VW,), jnp.int32),
                       pltpu.VMEM((_VW,), jnp.int32),
                       pltpu.VMEM((_VW,), jnp.int32),
                       pltpu.VMEM((_NW * _HB,), jnp.int32),
                       pltpu.VMEM((_VW, 64), f32),
                       pltpu.VMEM((_ZR, 64), f32),
                       pltpu.VMEM((_ACCR,), f32),
                       pltpu.VMEM((_ACCR,), jnp.int32),
                       pltpu.VMEM_SHARED((_ACCR, 64), f32),
                       pltpu.VMEM_SHARED((_ACCR,), f32),
                       pltpu.SemaphoreType.DMA],
        compiler_params=pltpu.CompilerParams(use_tc_tiling_on_sc=False,
                                             needs_layout_passes=False),
    )(packed, hist, m)

    psum = pl.pallas_call(
        _k4_body,
        grid=(_NP // _NBK,),
        in_specs=[pl.BlockSpec((_NBK, 16), lambda i: (i, 0)),
                  pl.BlockSpec((_NBK, 64), lambda i: (i, 0)),
                  pl.BlockSpec((_NBK, 1), lambda i: (i, 0)),
                  pl.BlockSpec((_NBK, 1), lambda i: (i, 0)),
                  pl.BlockSpec((216, 64), lambda i: (0, 0))],
        out_specs=pl.BlockSpec((_G, 128), lambda i: (0, 0)),
        out_shape=jax.ShapeDtypeStruct((_G, 128), f32),
    )(x16, msum, mcnt.reshape(_NP, 1), batchf, wcat2)

    out = pl.pallas_call(
        _k5_body,
        grid=(1,),
        in_specs=[pl.BlockSpec((_G, 128), lambda i: (0, 0)),
                  pl.BlockSpec((200, 64), lambda i: (0, 0)),
                  pl.BlockSpec((64, 32), lambda i: (0, 0)),
                  pl.BlockSpec((8, 32), lambda i: (0, 0))],
        out_specs=pl.BlockSpec((_G, 128), lambda i: (0, 0)),
        out_shape=jax.ShapeDtypeStruct((_G, 128), f32),
    )(psum, gwcat, ht, hb)

    return (out[:, 0:1], out[:, 1:5], out[:, 5:9], out[:, 9:13],
            out[:, 13:17], out[:, 17:21], out[:, 21:25])


# speedup vs baseline: 1.3049x; 1.0741x over previous
import jax
import jax.numpy as jnp
from jax import lax
from jax.experimental import pallas as pl
from jax.experimental.pallas import tpu as pltpu, tpu_sc as plsc

_N = 100000
_E = 1600000
_G = 256
_LEAK = 0.1

_NP = 100352
_NB = 98
_BK = 1024
_NW = 32
_ES = _E // _NW
_RS = 50688
_PACKED = _NW * _RS + 512
_HB = 112
_MROWS = _E + 64
_ACCR = 1152
_ZR = _ACCR // 16
_GW = 2000
_CW = 2000
_VW = 128
_EB = 8000
_NBK = 1024

_BN_S = (1.0 + 1e-5) ** -0.5


def _leaky(v):
    return jnp.where(v > 0, v, _LEAK * v)


def _wid():
    return lax.axis_index("s") * 2 + lax.axis_index("c")


def _k1_body(x16, row, col, gsrc, gdst, idx_v, rows_v, sem):
    w = _wid()
    base0 = w * _ES

    def win(i, carry):
        base = pl.multiple_of(base0 + i * _GW, 8)
        pltpu.sync_copy(row.at[pl.ds(base, _GW)], idx_v)
        pltpu.async_copy(x16.at[idx_v], rows_v, sem).wait()
        pltpu.sync_copy(rows_v, gsrc.at[pl.ds(base, _GW)])
        pltpu.sync_copy(col.at[pl.ds(base, _GW)], idx_v)
        pltpu.async_copy(x16.at[idx_v], rows_v, sem).wait()
        pltpu.sync_copy(rows_v, gdst.at[pl.ds(base, _GW)])
        return carry

    lax.fori_loop(0, _ES // _GW, win, 0)


def _st1(ref, o, val):
    idx = jnp.broadcast_to(o, (16,))
    v = jnp.broadcast_to(val, (16,))
    plsc.store_scatter(ref, [idx], v, mask=lax.iota(jnp.int32, 16) == 0)


def _k3a_body(col, packed, hist, colv, hstage, outw, histv, curv, basev):
    w = _wid()
    base0 = w * _ES

    def zero(i, carry):
        histv[i] = 0
        return carry

    lax.fori_loop(0, _HB, zero, 0)

    def hwin(i, carry):
        pltpu.sync_copy(col.at[pl.ds(pl.multiple_of(base0 + i * _CW, 8), _CW)], colv)

        def inner(v, c2):
            bv = colv[pl.ds(v * 16, 16)] >> 10
            for l in range(16):
                b = bv[l]
                histv[b] = histv[b] + 1
            return c2

        lax.fori_loop(0, _CW // 16, inner, 0)
        return carry

    lax.fori_loop(0, _ES // _CW, hwin, 0)

    def pfx(b, run):
        basev[b] = run
        curv[b] = run
        _st1(hstage, b, histv[b])
        return run + ((histv[b] + 7) & (-8))

    lax.fori_loop(0, _NB, pfx, 0)

    def pwin(i, carry):
        wbase = base0 + i * _CW
        pltpu.sync_copy(col.at[pl.ds(pl.multiple_of(wbase, 8), _CW)], colv)

        def inner(v, c2):
            cv = colv[pl.ds(v * 16, 16)]
            wordv = (((wbase + v * 16 + lax.iota(jnp.int32, 16)) << 11)
                     | (cv & 1023))
            bv = cv >> 10
            for l in range(16):
                b = bv[l]
                o = curv[b]
                _st1(outw, o, wordv[l])
                curv[b] = o + 1
            return c2

        lax.fori_loop(0, _CW // 16, inner, 0)
        return carry

    lax.fori_loop(0, _ES // _CW, pwin, 0)

    dump = ((_E + w) << 11) | 1024

    def pad(b, carry):
        end = basev[b] + ((histv[b] + 7) & (-8))

        def fill(k2, c2):
            _st1(outw, k2, dump)
            return c2

        lax.fori_loop(curv[b], end, fill, 0)
        return carry

    lax.fori_loop(0, _NB, pad, 0)

    pltpu.sync_copy(outw, packed.at[pl.ds(w * _RS, _RS)])
    pltpu.sync_copy(hstage, hist.at[pl.ds(w * _HB, _HB)])


def _k3b_body(packed, hist, m, msum, mcnt,
              chunkv, idxv, offv, histv, mrows, zrow, cntp, idv,
              acc_sh, cnt_sh, sem):
    c = lax.axis_index("c")
    sid = lax.axis_index("s")
    pltpu.sync_copy(hist, histv)
    dump_eid = _E + _wid()
    dump_row = _BK + sid
    zero16 = jnp.zeros((16,), jnp.float32)
    iot16 = lax.iota(jnp.int32, 16)

    def initbufs(r, carry):
        zrow[r, pl.ds(0, 16)] = zero16
        zrow[r, pl.ds(16, 16)] = zero16
        zrow[r, pl.ds(32, 16)] = zero16
        zrow[r, pl.ds(48, 16)] = zero16
        return carry

    lax.fori_loop(0, _ZR, initbufs, 0)

    def initid(r, carry):
        idv[pl.ds(r * 16, 16)] = iot16 + r * 16
        return carry

    lax.fori_loop(0, _ACCR // 16, initid, 0)

    def bucket(k, carry):
        b = c + 2 * k
        zb = pl.multiple_of(sid * (_ACCR // 16), 8)
        pltpu.sync_copy(zrow.at[pl.ds(0, _ACCR // 16)],
                        acc_sh.at[pl.ds(zb, _ACCR // 16)])
        def zcnt(r, c2):
            cntp[pl.ds(r * 16, 16)] = zero16
            return c2

        lax.fori_loop(0, _ACCR // 16, zcnt, 0)
        pltpu.sync_copy(cntp.at[pl.ds(0, _ACCR // 16)],
                        cnt_sh.at[pl.ds(zb, _ACCR // 16)])
        plsc.subcore_barrier()

        for j in range(2):
            t = sid + 16 * j
            iot = lax.iota(jnp.int32, 16)
            segoff = 0
            seglen = 0
            for vi in range(_HB // 16):
                hv = histv[pl.ds(t * _HB + vi * 16, 16)]
                idx = iot + vi * 16
                r8 = (hv + 7) & (-8)
                segoff = segoff + jnp.sum(jnp.where(idx < b, r8, 0))
                seglen = seglen + jnp.sum(jnp.where(idx == b, r8, 0))
            segstart = pl.multiple_of(t * _RS + segoff, 8)
            nwin = (seglen + _VW - 1) // _VW

            def win(kk, c4):
                s0 = pl.multiple_of(segstart + kk * _VW, 8)
                pltpu.sync_copy(packed.at[pl.ds(s0, _VW)], chunkv)
                limit = seglen - kk * _VW
                for i in range(_VW // 16):
                    wv = plsc.bitcast(chunkv[pl.ds(i * 16, 16)], jnp.uint32)
                    pos = lax.iota(jnp.int32, 16) + (i * 16)
                    realm = pos < limit
                    eidv = jnp.where(
                        realm,
                        plsc.bitcast(wv >> jnp.uint32(11), jnp.int32),
                        dump_eid)
                    ofv = jnp.where(
                        realm,
                        plsc.bitcast(wv & jnp.uint32(2047), jnp.int32),
                        dump_row)
                    plsc.addupdate_scatter(cntp, [ofv],
                                           jnp.ones((16,), jnp.float32))
                    idxv[pl.ds(i * 16, 16)] = eidv
                    offv[pl.ds(i * 16, 16)] = ofv
                pltpu.async_copy(m.at[idxv], mrows, sem).wait()
                pltpu.sync_copy(mrows, acc_sh.at[offv], add=True)
                return c4

            lax.fori_loop(0, nwin, win, 0)

        pltpu.sync_copy(cntp, cnt_sh.at[idv], add=True)
        plsc.subcore_barrier()
        wb = pl.multiple_of(sid * (_BK // 16), 8)
        pltpu.sync_copy(acc_sh.at[pl.ds(wb, _BK // 16)],
                        msum.at[pl.ds(b * _BK + wb, _BK // 16)])
        pltpu.sync_copy(cnt_sh.at[pl.ds(wb, _BK // 16)],
                        mcnt.at[pl.ds(b * _BK + wb, _BK // 16)])
        plsc.subcore_barrier()
        return carry

    lax.fori_loop(0, _NB // 2, bucket, 0)


def _k2_body(gs, gd, ea, wc, mo):
    w = wc[...]
    a = jnp.dot(gs[...], w[0:16], preferred_element_type=jnp.float32)
    a = a + jnp.dot(gd[...], w[16:32], preferred_element_type=jnp.float32)
    a = a + jnp.dot(ea[...], w[32:44], preferred_element_type=jnp.float32)
    a = _leaky(a + w[384:385])
    a = _leaky(jnp.dot(a, w[48:112], preferred_element_type=jnp.float32)
               + w[385:386])
    e = jnp.dot(a, w[112:176], preferred_element_type=jnp.float32) + w[386:387]
    h = jnp.dot(gs[...], w[176:192], preferred_element_type=jnp.float32)
    h = h + jnp.dot(e, w[192:256], preferred_element_type=jnp.float32)
    h = _leaky(h + w[387:388])
    h = _leaky(jnp.dot(h, w[256:320], preferred_element_type=jnp.float32)
               + w[388:389])
    mo[...] = (jnp.dot(h, w[320:384], preferred_element_type=jnp.float32)
               + w[389:390])


def _k4_body(xb, ms, mc, bf, wc, out):
    i = pl.program_id(0)
    w = wc[...]
    cntc = jnp.maximum(mc[...][:, 0:1], 1.0)
    agg = ms[...] / cntc
    h = jnp.dot(xb[...], w[0:16], preferred_element_type=jnp.float32)
    h = h + jnp.dot(agg, w[16:80], preferred_element_type=jnp.float32)
    h = _leaky(h + w[208:209])
    h = _leaky(jnp.dot(h, w[80:144], preferred_element_type=jnp.float32)
               + w[209:210])
    xn = jnp.dot(h, w[144:208], preferred_element_type=jnp.float32) + w[210:211]
    oh = (bf[...] ==
          lax.broadcasted_iota(jnp.int32, (_NBK, _G), 1).astype(jnp.float32))
    oh = oh.astype(jnp.float32)
    xaug = jnp.concatenate(
        [xn, jnp.ones((_NBK, 1), jnp.float32),
         jnp.zeros((_NBK, 63), jnp.float32)], axis=1)
    contrib = lax.dot_general(oh, xaug, (((0,), (0,)), ((), ())),
                              preferred_element_type=jnp.float32)

    @pl.when(i == 0)
    def _():
        out[...] = jnp.zeros_like(out)

    out[...] = out[...] + contrib


def _k5_body(ps, gw, ht, hb, out):
    p = ps[...]
    wg = gw[...]
    u = p[:, 0:64] / jnp.maximum(p[:, 64:65], 1.0)
    u = _leaky(jnp.dot(u, wg[0:64], preferred_element_type=jnp.float32)
               + wg[192:193])
    u = _leaky(jnp.dot(u, wg[64:128], preferred_element_type=jnp.float32)
               + wg[193:194])
    u = jnp.dot(u, wg[128:192], preferred_element_type=jnp.float32) + wg[194:195]
    z = jnp.dot(u, ht[...], preferred_element_type=jnp.float32) + hb[...][0:1]
    cols = [jax.nn.sigmoid(z[:, 0:1])]
    for gidx in range(6):
        zg = z[:, 1 + 4 * gidx:5 + 4 * gidx]
        zm = jnp.max(zg, axis=1, keepdims=True)
        ez = jnp.exp(zg - zm)
        cols.append(ez / jnp.sum(ez, axis=1, keepdims=True))
    cols.append(jnp.zeros((_G, 128 - 25), jnp.float32))
    out[...] = jnp.concatenate(cols, axis=1)


def _fold(params):
    out = []
    for (g, b, wt, c) in params:
        wl = (wt * (g * _BN_S)[None, :]).T
        out.append((wl, b @ wt.T + c))
    return out


def _z16(a):
    return jnp.pad(a, ((0, 16 - a.shape[0]), (0, 0)))


def kernel(x, edge_index, edge_attr, batch, edge_mlp, node_mlp1, node_mlp2,
           global_mlp, preds):
    f32 = jnp.float32
    row = edge_index[0]
    col = edge_index[1]
    x16 = jnp.pad(x, ((0, _NP - _N), (0, 16 - x.shape[1])))
    batchf = jnp.pad(batch.astype(f32), ((0, _NP - _N),),
                     constant_values=300.0).reshape(_NP, 1)

    em = _fold(edge_mlp)
    n1 = _fold(node_mlp1)
    n2 = _fold(node_mlp2)
    gl = _fold(global_mlp)
    (w1, c1), (w2, c2), (w3, c3) = em
    (b1, d1), (b2, d2), (b3, d3) = n1
    wcat = jnp.concatenate([
        _z16(w1[0:9]), _z16(w1[9:18]), w1[18:30], jnp.zeros((4, 64), f32),
        w2, w3, _z16(b1[0:9]), b1[9:73], b2, b3,
        c1[None], c2[None], c3[None], d1[None], d2[None], d3[None],
        jnp.zeros((2, 64), f32)], axis=0)
    (a1, e1), (a2, e2), (a3, e3) = n2
    wcat2 = jnp.concatenate([
        _z16(a1[0:9]), a1[9:73], a2, a3,
        e1[None], e2[None], e3[None], jnp.zeros((5, 64), f32)], axis=0)
    (g1, f1), (g2, f2), (g3, f3) = gl
    gwcat = jnp.concatenate([
        g1, g2, g3, f1[None], f2[None], f3[None],
        jnp.zeros((5, 64), f32)], axis=0)
    wh = jnp.concatenate([p[0] for p in preds], axis=0)
    ht = jnp.pad(wh.T, ((0, 0), (0, 7)))
    hbv = jnp.pad(jnp.concatenate([p[1] for p in preds]), ((0, 7),))
    hb = jnp.pad(hbv[None], ((0, 7), (0, 0)))

    mesh = plsc.VectorSubcoreMesh(core_axis_name="c", subcore_axis_name="s")


    packed, hist = pl.kernel(
        _k3a_body,
        out_type=(jax.ShapeDtypeStruct((_PACKED,), jnp.int32),
                  jax.ShapeDtypeStruct((_NW * _HB,), jnp.int32)),
        mesh=mesh,
        scratch_types=[pltpu.VMEM((_CW,), jnp.int32),
                       pltpu.VMEM((_HB,), jnp.int32),
                       pltpu.VMEM((_RS,), jnp.int32),
                       pltpu.SMEM((_HB,), jnp.int32),
                       pltpu.SMEM((_HB,), jnp.int32),
                       pltpu.SMEM((_HB,), jnp.int32)],
        compiler_params=pltpu.CompilerParams(needs_layout_passes=False),
    )(col)

    gsrc, gdst = pl.kernel(
        _k1_body,
        out_type=(jax.ShapeDtypeStruct((_E, 16), f32),
                  jax.ShapeDtypeStruct((_E, 16), f32)),
        mesh=mesh,
        scratch_types=[pltpu.VMEM((_GW,), jnp.int32),
                       pltpu.VMEM((_GW, 16), f32),
                       pltpu.SemaphoreType.DMA],
        compiler_params=pltpu.CompilerParams(use_tc_tiling_on_sc=False),
    )(x16, row, col)

    m = pl.pallas_call(
        _k2_body,
        grid=(_E // _EB,),
        in_specs=[pl.BlockSpec((_EB, 16), lambda i: (i, 0)),
                  pl.BlockSpec((_EB, 16), lambda i: (i, 0)),
                  pl.BlockSpec((_EB, 12), lambda i: (i, 0)),
                  pl.BlockSpec((392, 64), lambda i: (0, 0))],
        out_specs=pl.BlockSpec((_EB, 64), lambda i: (i, 0)),
        out_shape=jax.ShapeDtypeStruct((_MROWS, 64), f32),
    )(gsrc, gdst, edge_attr, wcat)

    msum, mcnt = pl.kernel(
        _k3b_body,
        out_type=(jax.ShapeDtypeStruct((_NP, 64), f32),
                  jax.ShapeDtypeStruct((_NP,), f32)),
        mesh=mesh,
        scratch_types=[pltpu.VMEM((_VW,), jnp.int32),
                       pltpu.VMEM((_VW,), jnp.int32),
                       pltpu.VMEM((_VW,), jnp.int32),
                       pltpu.VMEM((_NW * _HB,), jnp.int32),
                       pltpu.VMEM((_VW, 64), f32),
                       pltpu.VMEM((_ZR, 64), f32),
                       pltpu.VMEM((_ACCR,), f32),
                       pltpu.VMEM((_ACCR,), jnp.int32),
                       pltpu.VMEM_SHARED((_ACCR, 64), f32),
                       pltpu.VMEM_SHARED((_ACCR,), f32),
                       pltpu.SemaphoreType.DMA],
        compiler_params=pltpu.CompilerParams(use_tc_tiling_on_sc=False,
                                             needs_layout_passes=False),
    )(packed, hist, m)

    psum = pl.pallas_call(
        _k4_body,
        grid=(_NP // _NBK,),
        in_specs=[pl.BlockSpec((_NBK, 16), lambda i: (i, 0)),
                  pl.BlockSpec((_NBK, 64), lambda i: (i, 0)),
                  pl.BlockSpec((_NBK, 1), lambda i: (i, 0)),
                  pl.BlockSpec((_NBK, 1), lambda i: (i, 0)),
                  pl.BlockSpec((216, 64), lambda i: (0, 0))],
        out_specs=pl.BlockSpec((_G, 128), lambda i: (0, 0)),
        out_shape=jax.ShapeDtypeStruct((_G, 128), f32),
    )(x16, msum, mcnt.reshape(_NP, 1), batchf, wcat2)

    out = pl.pallas_call(
        _k5_body,
        grid=(1,),
        in_specs=[pl.BlockSpec((_G, 128), lambda i: (0, 0)),
                  pl.BlockSpec((200, 64), lambda i: (0, 0)),
                  pl.BlockSpec((64, 32), lambda i: (0, 0)),
                  pl.BlockSpec((8, 32), lambda i: (0, 0))],
        out_specs=pl.BlockSpec((_G, 128), lambda i: (0, 0)),
        out_shape=jax.ShapeDtypeStruct((_G, 128), f32),
    )(psum, gwcat, ht, hb)

    return (out[:, 0:1], out[:, 1:5], out[:, 5:9], out[:, 9:13],
            out[:, 13:17], out[:, 17:21], out[:, 21:25])
